# baseline (speedup 1.0000x reference)
import functools

import jax
import jax.numpy as jnp
from jax import lax
from jax.experimental import pallas as pl
from jax.experimental.pallas import tpu as pltpu
from jax.experimental.pallas import tpu_sc as plsc

N = 10000
E = 320000
D = 128
H = 128
C = 10
G = 128

NC = 2
NS = 16
NW = NC * NS
CHUNK = 128
NCH = 80
EP = NW * NCH * CHUNK
NPAD = N + 112
STRIPE = NPAD // NS

BLK = 2000
NBLK = N // BLK

def _deg_kernel_build():
  return functools.partial(
    pl.kernel,
    out_type=jax.ShapeDtypeStruct((NC, NPAD, 128), jnp.float32),
    scratch_types=[
        pltpu.VMEM((2, CHUNK), jnp.int32),
        pltpu.VMEM((2, CHUNK), jnp.int32),
        pltpu.VMEM((CHUNK, 128), jnp.float32),
        pltpu.VMEM_SHARED((NPAD, 128), jnp.float32),
        pltpu.SemaphoreType.DMA,
        pltpu.SemaphoreType.DMA,
    ],
    mesh=plsc.VectorSubcoreMesh(core_axis_name="c", subcore_axis_name="s",
                                num_cores=NC, num_subcores=NS),
  )(_deg_body)


def _deg_body(ed4, zrows, ones128, out, idx_a, idx_b, ones_v, spm,
              sem_a, sem_b):
    c = lax.axis_index("c")
    s = lax.axis_index("s")
    w = s * NC + c
    pltpu.sync_copy(zrows, spm.at[pl.ds(s * STRIPE, STRIPE)])
    pltpu.sync_copy(ones128, ones_v)
    plsc.subcore_barrier()

    pltpu.sync_copy(ed4.at[w, 0], idx_a)
    pltpu.async_copy(ed4.at[w, 1], idx_b, sem_b)

    def step(p, carry):
        j = 2 * p
        pltpu.sync_copy(ones_v, spm.at[idx_a.at[1]], add=True)
        pltpu.make_async_copy(ed4.at[w, j + 1], idx_b, sem_b).wait()

        @pl.when(p < NCH // 2 - 1)
        def _():
            pltpu.async_copy(ed4.at[w, j + 2], idx_a, sem_a)

        pltpu.sync_copy(ones_v, spm.at[idx_b.at[1]], add=True)

        @pl.when(p < NCH // 2 - 1)
        def _():
            pltpu.make_async_copy(ed4.at[w, j + 2], idx_a, sem_a).wait()
            pltpu.async_copy(ed4.at[w, j + 3], idx_b, sem_b)

        return carry

    lax.fori_loop(0, NCH // 2, step, 0)
    plsc.subcore_barrier()
    pltpu.sync_copy(spm.at[pl.ds(s * STRIPE, STRIPE)],
                    out.at[c, pl.ds(s * STRIPE, STRIPE)])


def _agg_kernel_build():
  return functools.partial(
    pl.kernel,
    out_type=jax.ShapeDtypeStruct((NC, NPAD, H), jnp.float32),
    scratch_types=[
        pltpu.VMEM((2, 2, CHUNK), jnp.int32),
        pltpu.VMEM((2, 2, CHUNK), jnp.int32),
        pltpu.VMEM((CHUNK, H), jnp.float32),
        pltpu.VMEM((CHUNK, H), jnp.float32),
        pltpu.VMEM_SHARED((NPAD, H), jnp.float32),
        pltpu.SemaphoreType.DMA,
        pltpu.SemaphoreType.DMA,
        pltpu.SemaphoreType.DMA,
        pltpu.SemaphoreType.DMA,
    ],
    mesh=plsc.VectorSubcoreMesh(core_axis_name="c", subcore_axis_name="s",
                                num_cores=NC, num_subcores=NS),
  )(_agg_body)


def _agg_body(ed4, y_hbm, zrows, out,
              ip_a, ip_b, rows_a, rows_b, spm, sem_ia, sem_ib, sem_ga, sem_gb):
    c = lax.axis_index("c")
    s = lax.axis_index("s")
    w = s * NC + c
    Q = NCH // 4
    pltpu.sync_copy(zrows, spm.at[pl.ds(s * STRIPE, STRIPE)])
    plsc.subcore_barrier()

    pltpu.sync_copy(ed4.at[w, pl.ds(0, 2)], ip_a)
    pltpu.async_copy(ed4.at[w, pl.ds(2, 2)], ip_b, sem_ib)
    pltpu.async_copy(y_hbm.at[ip_a.at[0, 0]], rows_a, sem_ga)

    def quad(q, carry):
        b = 4 * q
        pltpu.make_async_copy(y_hbm.at[ip_a.at[0, 0]], rows_a, sem_ga).wait()
        pltpu.async_copy(y_hbm.at[ip_a.at[1, 0]], rows_b, sem_gb)
        pltpu.sync_copy(rows_a, spm.at[ip_a.at[0, 1]], add=True)

        pltpu.make_async_copy(ed4.at[w, pl.ds(0, 2)], ip_b, sem_ib).wait()
        pltpu.async_copy(y_hbm.at[ip_b.at[0, 0]], rows_a, sem_ga)
        pltpu.make_async_copy(y_hbm.at[ip_a.at[1, 0]], rows_b, sem_gb).wait()
        pltpu.sync_copy(rows_b, spm.at[ip_a.at[1, 1]], add=True)

        @pl.when(q < Q - 1)
        def _():
            pltpu.async_copy(ed4.at[w, pl.ds(b + 4, 2)], ip_a, sem_ia)

        pltpu.async_copy(y_hbm.at[ip_b.at[1, 0]], rows_b, sem_gb)
        pltpu.make_async_copy(y_hbm.at[ip_b.at[0, 0]], rows_a, sem_ga).wait()
        pltpu.sync_copy(rows_a, spm.at[ip_b.at[0, 1]], add=True)

        @pl.when(q < Q - 1)
        def _():
            pltpu.make_async_copy(ed4.at[w, pl.ds(0, 2)], ip_a, sem_ia).wait()
            pltpu.async_copy(y_hbm.at[ip_a.at[0, 0]], rows_a, sem_ga)

        pltpu.make_async_copy(y_hbm.at[ip_b.at[1, 0]], rows_b, sem_gb).wait()
        pltpu.sync_copy(rows_b, spm.at[ip_b.at[1, 1]], add=True)

        @pl.when(q < Q - 1)
        def _():
            pltpu.async_copy(ed4.at[w, pl.ds(b + 6, 2)], ip_b, sem_ib)

        return carry

    lax.fori_loop(0, Q, quad, 0)

    plsc.subcore_barrier()
    pltpu.sync_copy(spm.at[pl.ds(s * STRIPE, STRIPE)],
                    out.at[c, pl.ds(s * STRIPE, STRIPE)])


def _dinv_block(dp_ref):
    deg = 1.0 + dp_ref[0, :, 0:1] + dp_ref[1, :, 0:1]
    return lax.rsqrt(deg)


def _tc_mm_body(x_ref, w_ref, y_ref):
    y_ref[...] = jnp.dot(x_ref[...], w_ref[...],
                         preferred_element_type=jnp.float32)


def _tc_mm(x, W1):
    return pl.pallas_call(
        _tc_mm_body,
        grid=(NBLK,),
        in_specs=[
            pl.BlockSpec((BLK, D), lambda i: (i, 0)),
            pl.BlockSpec((D, H), lambda i: (0, 0)),
        ],
        out_specs=pl.BlockSpec((BLK, H), lambda i: (i, 0)),
        out_shape=jax.ShapeDtypeStruct((N, H), jnp.float32),
    )(x, W1)


def _tc_scale_body(xw_ref, dp_ref, y_ref):
    y_ref[...] = xw_ref[...] * _dinv_block(dp_ref)


def _tc_scale(xw, degp):
    return pl.pallas_call(
        _tc_scale_body,
        grid=(NBLK,),
        in_specs=[
            pl.BlockSpec((BLK, H), lambda i: (i, 0)),
            pl.BlockSpec((2, BLK, 8), lambda i: (0, i, 0)),
        ],
        out_specs=pl.BlockSpec((BLK, H), lambda i: (i, 0)),
        out_shape=jax.ShapeDtypeStruct((N, H), jnp.float32),
    )(xw, degp)


def _tc_b_body(ap_ref, y1_ref, dp_ref, w_ref, b_ref, y2_ref):
    dinv = _dinv_block(dp_ref)
    pre = dinv * (ap_ref[0] + ap_ref[1] + y1_ref[...]) + b_ref[...]
    h = jnp.maximum(pre, 0.0)
    hw = jnp.dot(h, w_ref[...], preferred_element_type=jnp.float32)
    y2_ref[...] = hw * dinv


def _tc_b(aggp, y1, degp, W2, b1):
    return pl.pallas_call(
        _tc_b_body,
        grid=(NBLK,),
        in_specs=[
            pl.BlockSpec((2, BLK, H), lambda i: (0, i, 0)),
            pl.BlockSpec((BLK, H), lambda i: (i, 0)),
            pl.BlockSpec((2, BLK, 8), lambda i: (0, i, 0)),
            pl.BlockSpec((H, H), lambda i: (0, 0)),
            pl.BlockSpec((1, H), lambda i: (0, 0)),
        ],
        out_specs=pl.BlockSpec((BLK, H), lambda i: (i, 0)),
        out_shape=jax.ShapeDtypeStruct((N, H), jnp.float32),
    )(aggp, y1, degp, W2, b1)


def _tc_c_body(ap_ref, y2_ref, dp_ref, b_ref, wc_ref, bc_ref,
               batch_ref, logits_ref, pooled_ref, pacc_ref, cacc_ref):
    i = pl.program_id(0)

    @pl.when(i == 0)
    def _():
        pacc_ref[...] = jnp.zeros_like(pacc_ref)
        cacc_ref[...] = jnp.zeros_like(cacc_ref)

    dinv = _dinv_block(dp_ref)
    pre = dinv * (ap_ref[0] + ap_ref[1] + y2_ref[...]) + b_ref[...]
    h2 = jnp.maximum(pre, 0.0)
    b = batch_ref[0, 0, :]
    gids = lax.broadcasted_iota(jnp.int32, (G, BLK), 0)
    onehot = (b[None, :] == gids).astype(jnp.float32)
    pacc_ref[...] += jnp.dot(onehot, h2, preferred_element_type=jnp.float32)
    cacc_ref[...] += jnp.dot(onehot, jnp.ones((BLK, 128), jnp.float32),
                             preferred_element_type=jnp.float32)

    @pl.when(i == NBLK - 1)
    def _():
        cnt = jnp.maximum(cacc_ref[...], 1.0)
        pooled = pacc_ref[...] / cnt
        pooled_ref[...] = pooled
        logits_ref[...] = (
            jnp.dot(pooled, wc_ref[...], preferred_element_type=jnp.float32)
            + bc_ref[...])


def _tc_c(aggp, y2, degp, b2, WcP, bcP, batch3):
    return pl.pallas_call(
        _tc_c_body,
        grid=(NBLK,),
        in_specs=[
            pl.BlockSpec((2, BLK, H), lambda i: (0, i, 0)),
            pl.BlockSpec((BLK, H), lambda i: (i, 0)),
            pl.BlockSpec((2, BLK, 8), lambda i: (0, i, 0)),
            pl.BlockSpec((1, H), lambda i: (0, 0)),
            pl.BlockSpec((H, 128), lambda i: (0, 0)),
            pl.BlockSpec((1, 128), lambda i: (0, 0)),
            pl.BlockSpec((1, 1, BLK), lambda i: (i, 0, 0)),
        ],
        out_specs=[
            pl.BlockSpec((G, 128), lambda i: (0, 0)),
            pl.BlockSpec((G, H), lambda i: (0, 0)),
        ],
        out_shape=[
            jax.ShapeDtypeStruct((G, 128), jnp.float32),
            jax.ShapeDtypeStruct((G, H), jnp.float32),
        ],
        scratch_shapes=[
            pltpu.VMEM((G, H), jnp.float32),
            pltpu.VMEM((G, 128), jnp.float32),
        ],
    )(aggp, y2, degp, b2, WcP, bcP, batch3)


def kernel(x, edge_index, batch, W1, b1, W2, b2, Wc, bc):
    pad = EP - E
    src_p = jnp.concatenate([edge_index[0],
                             jnp.arange(pad, dtype=jnp.int32) % N])
    dst_p = jnp.concatenate([edge_index[1],
                             N + (jnp.arange(pad, dtype=jnp.int32) % 112)])
    src3 = src_p.reshape(NW, NCH, CHUNK)
    dst3 = dst_p.reshape(NW, NCH, CHUNK)
    ed4 = jnp.stack([src3, dst3], axis=2)

    ones128 = jnp.ones((CHUNK, 128), jnp.float32)
    zrows = jnp.zeros((STRIPE, H), jnp.float32)

    deg_k = _deg_kernel_build()
    agg_k = _agg_kernel_build()
    xw1 = _tc_mm(x, W1)
    degp = deg_k(ed4, zrows, ones128)[:, :, :8]
    y1 = _tc_scale(xw1, degp)
    agg1 = agg_k(ed4, y1, zrows)
    y2 = _tc_b(agg1, y1, degp, W2, b1.reshape(1, H))
    agg2 = agg_k(ed4, y2, zrows)

    WcP = jnp.pad(Wc, ((0, 0), (0, 128 - C)))
    bcP = jnp.pad(bc, (0, 128 - C)).reshape(1, 128)
    batch3 = batch.reshape(NBLK, 1, BLK)
    logitsP, pooled = _tc_c(agg2, y2, degp,
                            b2.reshape(1, H), WcP, bcP, batch3)
    return logitsP[:, :C], pooled


# speedup vs baseline: 1.0036x; 1.0036x over previous
import functools

import jax
import jax.numpy as jnp
from jax import lax
from jax.experimental import pallas as pl
from jax.experimental.pallas import tpu as pltpu
from jax.experimental.pallas import tpu_sc as plsc

N = 10000
E = 320000
D = 128
H = 128
C = 10
G = 128

NC = 2
NS = 16
NW = NC * NS
CHUNK = 128
NCH = 80
EP = NW * NCH * CHUNK
NPAD = N + 112
STRIPE = NPAD // NS

BLK = 5000
NBLK = N // BLK

def _deg_kernel_build():
  return functools.partial(
    pl.kernel,
    out_type=jax.ShapeDtypeStruct((NC, NPAD, 128), jnp.float32),
    scratch_types=[
        pltpu.VMEM((2, CHUNK), jnp.int32),
        pltpu.VMEM((2, CHUNK), jnp.int32),
        pltpu.VMEM((CHUNK, 128), jnp.float32),
        pltpu.VMEM_SHARED((NPAD, 128), jnp.float32),
        pltpu.SemaphoreType.DMA,
        pltpu.SemaphoreType.DMA,
    ],
    mesh=plsc.VectorSubcoreMesh(core_axis_name="c", subcore_axis_name="s",
                                num_cores=NC, num_subcores=NS),
  )(_deg_body)


def _deg_body(ed4, zrows, ones128, out, idx_a, idx_b, ones_v, spm,
              sem_a, sem_b):
    c = lax.axis_index("c")
    s = lax.axis_index("s")
    w = s * NC + c
    pltpu.sync_copy(zrows, spm.at[pl.ds(s * STRIPE, STRIPE)])
    pltpu.sync_copy(ones128, ones_v)
    plsc.subcore_barrier()

    pltpu.sync_copy(ed4.at[w, 0], idx_a)
    pltpu.async_copy(ed4.at[w, 1], idx_b, sem_b)

    def step(p, carry):
        j = 2 * p
        pltpu.sync_copy(ones_v, spm.at[idx_a.at[1]], add=True)
        pltpu.make_async_copy(ed4.at[w, j + 1], idx_b, sem_b).wait()

        @pl.when(p < NCH // 2 - 1)
        def _():
            pltpu.async_copy(ed4.at[w, j + 2], idx_a, sem_a)

        pltpu.sync_copy(ones_v, spm.at[idx_b.at[1]], add=True)

        @pl.when(p < NCH // 2 - 1)
        def _():
            pltpu.make_async_copy(ed4.at[w, j + 2], idx_a, sem_a).wait()
            pltpu.async_copy(ed4.at[w, j + 3], idx_b, sem_b)

        return carry

    lax.fori_loop(0, NCH // 2, step, 0)
    plsc.subcore_barrier()
    pltpu.sync_copy(spm.at[pl.ds(s * STRIPE, STRIPE)],
                    out.at[c, pl.ds(s * STRIPE, STRIPE)])


def _agg_kernel_build():
  return functools.partial(
    pl.kernel,
    out_type=jax.ShapeDtypeStruct((NC, NPAD, H), jnp.float32),
    scratch_types=[
        pltpu.VMEM((2, 2, CHUNK), jnp.int32),
        pltpu.VMEM((2, 2, CHUNK), jnp.int32),
        pltpu.VMEM((CHUNK, H), jnp.float32),
        pltpu.VMEM((CHUNK, H), jnp.float32),
        pltpu.VMEM_SHARED((NPAD, H), jnp.float32),
        pltpu.SemaphoreType.DMA,
        pltpu.SemaphoreType.DMA,
        pltpu.SemaphoreType.DMA,
        pltpu.SemaphoreType.DMA,
    ],
    mesh=plsc.VectorSubcoreMesh(core_axis_name="c", subcore_axis_name="s",
                                num_cores=NC, num_subcores=NS),
  )(_agg_body)


def _agg_body(ed4, y_hbm, zrows, out,
              ip_a, ip_b, rows_a, rows_b, spm, sem_ia, sem_ib, sem_ga, sem_gb):
    c = lax.axis_index("c")
    s = lax.axis_index("s")
    w = s * NC + c
    Q = NCH // 4
    pltpu.sync_copy(zrows, spm.at[pl.ds(s * STRIPE, STRIPE)])
    plsc.subcore_barrier()

    pltpu.sync_copy(ed4.at[w, pl.ds(0, 2)], ip_a)
    pltpu.async_copy(ed4.at[w, pl.ds(2, 2)], ip_b, sem_ib)
    pltpu.async_copy(y_hbm.at[ip_a.at[0, 0]], rows_a, sem_ga)

    def quad(q, carry):
        b = 4 * q
        pltpu.make_async_copy(y_hbm.at[ip_a.at[0, 0]], rows_a, sem_ga).wait()
        pltpu.async_copy(y_hbm.at[ip_a.at[1, 0]], rows_b, sem_gb)
        pltpu.sync_copy(rows_a, spm.at[ip_a.at[0, 1]], add=True)

        pltpu.make_async_copy(ed4.at[w, pl.ds(0, 2)], ip_b, sem_ib).wait()
        pltpu.async_copy(y_hbm.at[ip_b.at[0, 0]], rows_a, sem_ga)
        pltpu.make_async_copy(y_hbm.at[ip_a.at[1, 0]], rows_b, sem_gb).wait()
        pltpu.sync_copy(rows_b, spm.at[ip_a.at[1, 1]], add=True)

        @pl.when(q < Q - 1)
        def _():
            pltpu.async_copy(ed4.at[w, pl.ds(b + 4, 2)], ip_a, sem_ia)

        pltpu.async_copy(y_hbm.at[ip_b.at[1, 0]], rows_b, sem_gb)
        pltpu.make_async_copy(y_hbm.at[ip_b.at[0, 0]], rows_a, sem_ga).wait()
        pltpu.sync_copy(rows_a, spm.at[ip_b.at[0, 1]], add=True)

        @pl.when(q < Q - 1)
        def _():
            pltpu.make_async_copy(ed4.at[w, pl.ds(0, 2)], ip_a, sem_ia).wait()
            pltpu.async_copy(y_hbm.at[ip_a.at[0, 0]], rows_a, sem_ga)

        pltpu.make_async_copy(y_hbm.at[ip_b.at[1, 0]], rows_b, sem_gb).wait()
        pltpu.sync_copy(rows_b, spm.at[ip_b.at[1, 1]], add=True)

        @pl.when(q < Q - 1)
        def _():
            pltpu.async_copy(ed4.at[w, pl.ds(b + 6, 2)], ip_b, sem_ib)

        return carry

    lax.fori_loop(0, Q, quad, 0)

    plsc.subcore_barrier()
    pltpu.sync_copy(spm.at[pl.ds(s * STRIPE, STRIPE)],
                    out.at[c, pl.ds(s * STRIPE, STRIPE)])


def _dinv_block(dp_ref):
    deg = 1.0 + dp_ref[0, :, 0:1] + dp_ref[1, :, 0:1]
    return lax.rsqrt(deg)


def _tc_mm_body(x_ref, w_ref, y_ref):
    y_ref[...] = jnp.dot(x_ref[...], w_ref[...],
                         preferred_element_type=jnp.float32)


def _tc_mm(x, W1):
    return pl.pallas_call(
        _tc_mm_body,
        grid=(NBLK,),
        in_specs=[
            pl.BlockSpec((BLK, D), lambda i: (i, 0)),
            pl.BlockSpec((D, H), lambda i: (0, 0)),
        ],
        out_specs=pl.BlockSpec((BLK, H), lambda i: (i, 0)),
        out_shape=jax.ShapeDtypeStruct((N, H), jnp.float32),
    )(x, W1)


def _tc_scale_body(xw_ref, dp_ref, y_ref):
    y_ref[...] = xw_ref[...] * _dinv_block(dp_ref)


def _tc_scale(xw, degp):
    return pl.pallas_call(
        _tc_scale_body,
        grid=(NBLK,),
        in_specs=[
            pl.BlockSpec((BLK, H), lambda i: (i, 0)),
            pl.BlockSpec((2, BLK, 8), lambda i: (0, i, 0)),
        ],
        out_specs=pl.BlockSpec((BLK, H), lambda i: (i, 0)),
        out_shape=jax.ShapeDtypeStruct((N, H), jnp.float32),
    )(xw, degp)


def _tc_b_body(ap_ref, y1_ref, dp_ref, w_ref, b_ref, y2_ref):
    dinv = _dinv_block(dp_ref)
    pre = dinv * (ap_ref[0] + ap_ref[1] + y1_ref[...]) + b_ref[...]
    h = jnp.maximum(pre, 0.0)
    hw = jnp.dot(h, w_ref[...], preferred_element_type=jnp.float32)
    y2_ref[...] = hw * dinv


def _tc_b(aggp, y1, degp, W2, b1):
    return pl.pallas_call(
        _tc_b_body,
        grid=(NBLK,),
        in_specs=[
            pl.BlockSpec((2, BLK, H), lambda i: (0, i, 0)),
            pl.BlockSpec((BLK, H), lambda i: (i, 0)),
            pl.BlockSpec((2, BLK, 8), lambda i: (0, i, 0)),
            pl.BlockSpec((H, H), lambda i: (0, 0)),
            pl.BlockSpec((1, H), lambda i: (0, 0)),
        ],
        out_specs=pl.BlockSpec((BLK, H), lambda i: (i, 0)),
        out_shape=jax.ShapeDtypeStruct((N, H), jnp.float32),
    )(aggp, y1, degp, W2, b1)


def _tc_c_body(ap_ref, y2_ref, dp_ref, b_ref, wc_ref, bc_ref,
               batch_ref, logits_ref, pooled_ref, pacc_ref, cacc_ref):
    i = pl.program_id(0)

    @pl.when(i == 0)
    def _():
        pacc_ref[...] = jnp.zeros_like(pacc_ref)
        cacc_ref[...] = jnp.zeros_like(cacc_ref)

    dinv = _dinv_block(dp_ref)
    pre = dinv * (ap_ref[0] + ap_ref[1] + y2_ref[...]) + b_ref[...]
    h2 = jnp.maximum(pre, 0.0)
    b = batch_ref[0, 0, :]
    gids = lax.broadcasted_iota(jnp.int32, (G, BLK), 0)
    onehot = (b[None, :] == gids).astype(jnp.float32)
    pacc_ref[...] += jnp.dot(onehot, h2, preferred_element_type=jnp.float32)
    cacc_ref[...] += jnp.dot(onehot, jnp.ones((BLK, 128), jnp.float32),
                             preferred_element_type=jnp.float32)

    @pl.when(i == NBLK - 1)
    def _():
        cnt = jnp.maximum(cacc_ref[...], 1.0)
        pooled = pacc_ref[...] / cnt
        pooled_ref[...] = pooled
        logits_ref[...] = (
            jnp.dot(pooled, wc_ref[...], preferred_element_type=jnp.float32)
            + bc_ref[...])


def _tc_c(aggp, y2, degp, b2, WcP, bcP, batch3):
    return pl.pallas_call(
        _tc_c_body,
        grid=(NBLK,),
        in_specs=[
            pl.BlockSpec((2, BLK, H), lambda i: (0, i, 0)),
            pl.BlockSpec((BLK, H), lambda i: (i, 0)),
            pl.BlockSpec((2, BLK, 8), lambda i: (0, i, 0)),
            pl.BlockSpec((1, H), lambda i: (0, 0)),
            pl.BlockSpec((H, 128), lambda i: (0, 0)),
            pl.BlockSpec((1, 128), lambda i: (0, 0)),
            pl.BlockSpec((1, 1, BLK), lambda i: (i, 0, 0)),
        ],
        out_specs=[
            pl.BlockSpec((G, 128), lambda i: (0, 0)),
            pl.BlockSpec((G, H), lambda i: (0, 0)),
        ],
        out_shape=[
            jax.ShapeDtypeStruct((G, 128), jnp.float32),
            jax.ShapeDtypeStruct((G, H), jnp.float32),
        ],
        scratch_shapes=[
            pltpu.VMEM((G, H), jnp.float32),
            pltpu.VMEM((G, 128), jnp.float32),
        ],
    )(aggp, y2, degp, b2, WcP, bcP, batch3)


def kernel(x, edge_index, batch, W1, b1, W2, b2, Wc, bc):
    pad = EP - E
    src_p = jnp.concatenate([edge_index[0],
                             jnp.arange(pad, dtype=jnp.int32) % N])
    dst_p = jnp.concatenate([edge_index[1],
                             N + (jnp.arange(pad, dtype=jnp.int32) % 112)])
    src3 = src_p.reshape(NW, NCH, CHUNK)
    dst3 = dst_p.reshape(NW, NCH, CHUNK)
    ed4 = jnp.stack([src3, dst3], axis=2)

    ones128 = jnp.ones((CHUNK, 128), jnp.float32)
    zrows = jnp.zeros((STRIPE, H), jnp.float32)

    deg_k = _deg_kernel_build()
    agg_k = _agg_kernel_build()
    degp = deg_k(ed4, zrows, ones128)[:, :, :8]
    y1 = _tc_scale(_tc_mm(x, W1), degp)
    agg1 = agg_k(ed4, y1, zrows)
    y2 = _tc_b(agg1, y1, degp, W2, b1.reshape(1, H))
    agg2 = agg_k(ed4, y2, zrows)

    WcP = jnp.pad(Wc, ((0, 0), (0, 128 - C)))
    bcP = jnp.pad(bc, (0, 128 - C)).reshape(1, 128)
    batch3 = batch.reshape(NBLK, 1, BLK)
    logitsP, pooled = _tc_c(agg2, y2, degp,
                            b2.reshape(1, H), WcP, bcP, batch3)
    return logitsP[:, :C], pooled


# speedup vs baseline: 1.0078x; 1.0042x over previous
import functools

import jax
import jax.numpy as jnp
from jax import lax
from jax.experimental import pallas as pl
from jax.experimental.pallas import tpu as pltpu
from jax.experimental.pallas import tpu_sc as plsc

N = 10000
E = 320000
D = 128
H = 128
C = 10
G = 128

NC = 2
NS = 16
NW = NC * NS
CHUNK = 128
NCH = 80
EP = NW * NCH * CHUNK
NPAD = N + 112
STRIPE = NPAD // NS

BLK = 2000
NBLK = N // BLK

def _deg_kernel_build():
  return functools.partial(
    pl.kernel,
    out_type=jax.ShapeDtypeStruct((NC, NPAD, 128), jnp.float32),
    scratch_types=[
        pltpu.VMEM((2, CHUNK), jnp.int32),
        pltpu.VMEM((2, CHUNK), jnp.int32),
        pltpu.VMEM((CHUNK, 128), jnp.float32),
        pltpu.VMEM_SHARED((NPAD, 128), jnp.float32),
        pltpu.SemaphoreType.DMA,
        pltpu.SemaphoreType.DMA,
    ],
    mesh=plsc.VectorSubcoreMesh(core_axis_name="c", subcore_axis_name="s",
                                num_cores=NC, num_subcores=NS),
  )(_deg_body)


def _deg_body(ed4, zrows, ones128, out, idx_a, idx_b, ones_v, spm,
              sem_a, sem_b):
    c = lax.axis_index("c")
    s = lax.axis_index("s")
    w = s * NC + c
    pltpu.sync_copy(zrows, spm.at[pl.ds(s * STRIPE, STRIPE)])
    pltpu.sync_copy(ones128, ones_v)
    plsc.subcore_barrier()

    pltpu.sync_copy(ed4.at[w, 0], idx_a)
    pltpu.async_copy(ed4.at[w, 1], idx_b, sem_b)

    def step(p, carry):
        j = 2 * p
        pltpu.sync_copy(ones_v, spm.at[idx_a.at[1]], add=True)
        pltpu.make_async_copy(ed4.at[w, j + 1], idx_b, sem_b).wait()

        @pl.when(p < NCH // 2 - 1)
        def _():
            pltpu.async_copy(ed4.at[w, j + 2], idx_a, sem_a)

        pltpu.sync_copy(ones_v, spm.at[idx_b.at[1]], add=True)

        @pl.when(p < NCH // 2 - 1)
        def _():
            pltpu.make_async_copy(ed4.at[w, j + 2], idx_a, sem_a).wait()
            pltpu.async_copy(ed4.at[w, j + 3], idx_b, sem_b)

        return carry

    lax.fori_loop(0, NCH // 2, step, 0)
    plsc.subcore_barrier()
    pltpu.sync_copy(spm.at[pl.ds(s * STRIPE, STRIPE)],
                    out.at[c, pl.ds(s * STRIPE, STRIPE)])


def _agg_kernel_build():
  return functools.partial(
    pl.kernel,
    out_type=jax.ShapeDtypeStruct((NC, NPAD, H), jnp.float32),
    scratch_types=[
        pltpu.VMEM((2, 2, CHUNK), jnp.int32),
        pltpu.VMEM((2, 2, CHUNK), jnp.int32),
        pltpu.VMEM((CHUNK, H), jnp.float32),
        pltpu.VMEM((CHUNK, H), jnp.float32),
        pltpu.VMEM_SHARED((NPAD, H), jnp.float32),
        pltpu.SemaphoreType.DMA,
        pltpu.SemaphoreType.DMA,
        pltpu.SemaphoreType.DMA,
        pltpu.SemaphoreType.DMA,
    ],
    mesh=plsc.VectorSubcoreMesh(core_axis_name="c", subcore_axis_name="s",
                                num_cores=NC, num_subcores=NS),
  )(_agg_body)


def _agg_body(ed4, y_hbm, zrows, out,
              ip_a, ip_b, rows_a, rows_b, spm, sem_ia, sem_ib, sem_ga, sem_gb):
    c = lax.axis_index("c")
    s = lax.axis_index("s")
    w = s * NC + c
    Q = NCH // 4
    pltpu.sync_copy(zrows, spm.at[pl.ds(s * STRIPE, STRIPE)])
    plsc.subcore_barrier()

    pltpu.sync_copy(ed4.at[w, pl.ds(0, 2)], ip_a)
    pltpu.async_copy(ed4.at[w, pl.ds(2, 2)], ip_b, sem_ib)
    pltpu.async_copy(y_hbm.at[ip_a.at[0, 0]], rows_a, sem_ga)

    def quad(q, carry):
        b = 4 * q
        pltpu.make_async_copy(y_hbm.at[ip_a.at[0, 0]], rows_a, sem_ga).wait()
        pltpu.async_copy(y_hbm.at[ip_a.at[1, 0]], rows_b, sem_gb)
        pltpu.sync_copy(rows_a, spm.at[ip_a.at[0, 1]], add=True)

        pltpu.make_async_copy(ed4.at[w, pl.ds(0, 2)], ip_b, sem_ib).wait()
        pltpu.async_copy(y_hbm.at[ip_b.at[0, 0]], rows_a, sem_ga)
        pltpu.make_async_copy(y_hbm.at[ip_a.at[1, 0]], rows_b, sem_gb).wait()
        pltpu.sync_copy(rows_b, spm.at[ip_a.at[1, 1]], add=True)

        @pl.when(q < Q - 1)
        def _():
            pltpu.async_copy(ed4.at[w, pl.ds(b + 4, 2)], ip_a, sem_ia)

        pltpu.async_copy(y_hbm.at[ip_b.at[1, 0]], rows_b, sem_gb)
        pltpu.make_async_copy(y_hbm.at[ip_b.at[0, 0]], rows_a, sem_ga).wait()
        pltpu.sync_copy(rows_a, spm.at[ip_b.at[0, 1]], add=True)

        @pl.when(q < Q - 1)
        def _():
            pltpu.make_async_copy(ed4.at[w, pl.ds(0, 2)], ip_a, sem_ia).wait()
            pltpu.async_copy(y_hbm.at[ip_a.at[0, 0]], rows_a, sem_ga)

        pltpu.make_async_copy(y_hbm.at[ip_b.at[1, 0]], rows_b, sem_gb).wait()
        pltpu.sync_copy(rows_b, spm.at[ip_b.at[1, 1]], add=True)

        @pl.when(q < Q - 1)
        def _():
            pltpu.async_copy(ed4.at[w, pl.ds(b + 6, 2)], ip_b, sem_ib)

        return carry

    lax.fori_loop(0, Q, quad, 0)

    plsc.subcore_barrier()
    pltpu.sync_copy(spm.at[pl.ds(s * STRIPE, STRIPE)],
                    out.at[c, pl.ds(s * STRIPE, STRIPE)])


def _dinv_block(dp_ref):
    deg = 1.0 + dp_ref[0, :, 0:1] + dp_ref[1, :, 0:1]
    return lax.rsqrt(deg)


def _tc_a_body(x_ref, dp_ref, w_ref, y_ref):
    dinv = _dinv_block(dp_ref)
    xw = jnp.dot(x_ref[...], w_ref[...], preferred_element_type=jnp.float32)
    y_ref[...] = xw * dinv


def _tc_a(x, degp, W1):
    return pl.pallas_call(
        _tc_a_body,
        grid=(NBLK,),
        in_specs=[
            pl.BlockSpec((BLK, D), lambda i: (i, 0)),
            pl.BlockSpec((2, BLK, 8), lambda i: (0, i, 0)),
            pl.BlockSpec((D, H), lambda i: (0, 0)),
        ],
        out_specs=pl.BlockSpec((BLK, H), lambda i: (i, 0)),
        out_shape=jax.ShapeDtypeStruct((N, H), jnp.float32),
    )(x, degp, W1)


def _tc_b_body(ap_ref, y1_ref, dp_ref, w_ref, b_ref, y2_ref):
    dinv = _dinv_block(dp_ref)
    pre = dinv * (ap_ref[0] + ap_ref[1] + y1_ref[...]) + b_ref[...]
    h = jnp.maximum(pre, 0.0)
    hw = jnp.dot(h, w_ref[...], preferred_element_type=jnp.float32)
    y2_ref[...] = hw * dinv


def _tc_b(aggp, y1, degp, W2, b1):
    return pl.pallas_call(
        _tc_b_body,
        grid=(NBLK,),
        in_specs=[
            pl.BlockSpec((2, BLK, H), lambda i: (0, i, 0)),
            pl.BlockSpec((BLK, H), lambda i: (i, 0)),
            pl.BlockSpec((2, BLK, 8), lambda i: (0, i, 0)),
            pl.BlockSpec((H, H), lambda i: (0, 0)),
            pl.BlockSpec((1, H), lambda i: (0, 0)),
        ],
        out_specs=pl.BlockSpec((BLK, H), lambda i: (i, 0)),
        out_shape=jax.ShapeDtypeStruct((N, H), jnp.float32),
    )(aggp, y1, degp, W2, b1)


def _tc_c_body(ap_ref, y2_ref, dp_ref, b_ref, wc_ref, bc_ref,
               batch_ref, logits_ref, pooled_ref, pacc_ref, cacc_ref):
    i = pl.program_id(0)

    @pl.when(i == 0)
    def _():
        pacc_ref[...] = jnp.zeros_like(pacc_ref)
        cacc_ref[...] = jnp.zeros_like(cacc_ref)

    dinv = _dinv_block(dp_ref)
    pre = dinv * (ap_ref[0] + ap_ref[1] + y2_ref[...]) + b_ref[...]
    h2 = jnp.maximum(pre, 0.0)
    b = batch_ref[0, 0, :]
    gids = lax.broadcasted_iota(jnp.int32, (G, BLK), 0)
    onehot = (b[None, :] == gids).astype(jnp.float32)
    pacc_ref[...] += jnp.dot(onehot, h2, preferred_element_type=jnp.float32)
    cacc_ref[...] += jnp.dot(onehot, jnp.ones((BLK, 128), jnp.float32),
                             preferred_element_type=jnp.float32)

    @pl.when(i == NBLK - 1)
    def _():
        cnt = jnp.maximum(cacc_ref[...], 1.0)
        pooled = pacc_ref[...] / cnt
        pooled_ref[...] = pooled
        logits_ref[...] = (
            jnp.dot(pooled, wc_ref[...], preferred_element_type=jnp.float32)
            + bc_ref[...])


def _tc_c(aggp, y2, degp, b2, WcP, bcP, batch3):
    return pl.pallas_call(
        _tc_c_body,
        grid=(NBLK,),
        in_specs=[
            pl.BlockSpec((2, BLK, H), lambda i: (0, i, 0)),
            pl.BlockSpec((BLK, H), lambda i: (i, 0)),
            pl.BlockSpec((2, BLK, 8), lambda i: (0, i, 0)),
            pl.BlockSpec((1, H), lambda i: (0, 0)),
            pl.BlockSpec((H, 128), lambda i: (0, 0)),
            pl.BlockSpec((1, 128), lambda i: (0, 0)),
            pl.BlockSpec((1, 1, BLK), lambda i: (i, 0, 0)),
        ],
        out_specs=[
            pl.BlockSpec((G, 128), lambda i: (0, 0)),
            pl.BlockSpec((G, H), lambda i: (0, 0)),
        ],
        out_shape=[
            jax.ShapeDtypeStruct((G, 128), jnp.float32),
            jax.ShapeDtypeStruct((G, H), jnp.float32),
        ],
        scratch_shapes=[
            pltpu.VMEM((G, H), jnp.float32),
            pltpu.VMEM((G, 128), jnp.float32),
        ],
    )(aggp, y2, degp, b2, WcP, bcP, batch3)


def kernel(x, edge_index, batch, W1, b1, W2, b2, Wc, bc):
    pad = EP - E
    src_p = jnp.concatenate([edge_index[0],
                             jnp.arange(pad, dtype=jnp.int32) % N])
    dst_p = jnp.concatenate([edge_index[1],
                             N + (jnp.arange(pad, dtype=jnp.int32) % 112)])
    src3 = src_p.reshape(NW, NCH, CHUNK)
    dst3 = dst_p.reshape(NW, NCH, CHUNK)
    ed4 = jnp.stack([src3, dst3], axis=2)

    ones128 = jnp.ones((CHUNK, 128), jnp.float32)
    zrows = jnp.zeros((STRIPE, H), jnp.float32)

    deg_k = _deg_kernel_build()
    agg_k = _agg_kernel_build()
    degp = deg_k(ed4, zrows, ones128)[:, :, :8]
    y1 = _tc_a(x, degp, W1)
    agg1 = agg_k(ed4, y1, zrows)
    y2 = _tc_b(agg1, y1, degp, W2, b1.reshape(1, H))
    agg2 = agg_k(ed4, y2, zrows)

    WcP = jnp.pad(Wc, ((0, 0), (0, 128 - C)))
    bcP = jnp.pad(bc, (0, 128 - C)).reshape(1, 128)
    batch3 = batch.reshape(NBLK, 1, BLK)
    logitsP, pooled = _tc_c(agg2, y2, degp,
                            b2.reshape(1, H), WcP, bcP, batch3)
    return logitsP[:, :C], pooled


# speedup vs baseline: 1.0241x; 1.0161x over previous
import functools

import jax
import jax.numpy as jnp
from jax import lax
from jax.experimental import pallas as pl
from jax.experimental.pallas import tpu as pltpu
from jax.experimental.pallas import tpu_sc as plsc

N = 10000
E = 320000
D = 128
H = 128
C = 10
G = 128

NC = 2
NS = 16
NW = NC * NS
CHUNK = 128
NCH = 80
EP = NW * NCH * CHUNK
NPAD = N + 112
STRIPE = NPAD // NS

BLK = 2000
NBLK = N // BLK

def _deg_kernel_build():
  return functools.partial(
    pl.kernel,
    out_type=jax.ShapeDtypeStruct((NC, NPAD, 128), jnp.float32),
    scratch_types=[
        pltpu.VMEM((2, CHUNK), jnp.int32),
        pltpu.VMEM((2, CHUNK), jnp.int32),
        pltpu.VMEM((CHUNK, 128), jnp.float32),
        pltpu.VMEM_SHARED((NPAD, 128), jnp.float32),
        pltpu.SemaphoreType.DMA,
        pltpu.SemaphoreType.DMA,
    ],
    mesh=plsc.VectorSubcoreMesh(core_axis_name="c", subcore_axis_name="s",
                                num_cores=NC, num_subcores=NS),
  )(_deg_body)


def _deg_body(ed4, zrows, ones128, out, idx_a, idx_b, ones_v, spm,
              sem_a, sem_b):
    c = lax.axis_index("c")
    s = lax.axis_index("s")
    w = s * NC + c
    pltpu.sync_copy(zrows, spm.at[pl.ds(s * STRIPE, STRIPE)])
    pltpu.sync_copy(ones128, ones_v)
    plsc.subcore_barrier()

    pltpu.sync_copy(ed4.at[w, 0], idx_a)
    pltpu.async_copy(ed4.at[w, 1], idx_b, sem_b)

    def step(p, carry):
        j = 2 * p
        pltpu.sync_copy(ones_v, spm.at[idx_a.at[1]], add=True)
        pltpu.make_async_copy(ed4.at[w, j + 1], idx_b, sem_b).wait()

        @pl.when(p < NCH // 2 - 1)
        def _():
            pltpu.async_copy(ed4.at[w, j + 2], idx_a, sem_a)

        pltpu.sync_copy(ones_v, spm.at[idx_b.at[1]], add=True)

        @pl.when(p < NCH // 2 - 1)
        def _():
            pltpu.make_async_copy(ed4.at[w, j + 2], idx_a, sem_a).wait()
            pltpu.async_copy(ed4.at[w, j + 3], idx_b, sem_b)

        return carry

    lax.fori_loop(0, NCH // 2, step, 0)
    plsc.subcore_barrier()
    pltpu.sync_copy(spm.at[pl.ds(s * STRIPE, STRIPE)],
                    out.at[c, pl.ds(s * STRIPE, STRIPE)])


def _agg_kernel_build():
  return functools.partial(
    pl.kernel,
    out_type=jax.ShapeDtypeStruct((NC, NPAD, H), jnp.float32),
    scratch_types=[
        pltpu.VMEM((2, 2, CHUNK), jnp.int32),
        pltpu.VMEM((2, 2, CHUNK), jnp.int32),
        pltpu.VMEM((CHUNK, H), jnp.float32),
        pltpu.VMEM((CHUNK, H), jnp.float32),
        pltpu.VMEM_SHARED((NPAD, H), jnp.float32),
        pltpu.SemaphoreType.DMA,
        pltpu.SemaphoreType.DMA,
        pltpu.SemaphoreType.DMA,
        pltpu.SemaphoreType.DMA,
    ],
    mesh=plsc.VectorSubcoreMesh(core_axis_name="c", subcore_axis_name="s",
                                num_cores=NC, num_subcores=NS),
  )(_agg_body)


def _agg_body(ed4, y_hbm, zrows, out,
              ip_a, ip_b, rows_a, rows_b, spm, sem_ia, sem_ib, sem_ga, sem_gb):
    c = lax.axis_index("c")
    s = lax.axis_index("s")
    w = s * NC + c
    Q = NCH // 4
    pltpu.sync_copy(zrows, spm.at[pl.ds(s * STRIPE, STRIPE)])
    plsc.subcore_barrier()

    pltpu.sync_copy(ed4.at[w, pl.ds(0, 2)], ip_a)
    pltpu.async_copy(ed4.at[w, pl.ds(2, 2)], ip_b, sem_ib)
    pltpu.async_copy(y_hbm.at[ip_a.at[0, 0]], rows_a, sem_ga)

    def quad(q, carry):
        b = 4 * q
        pltpu.async_copy(y_hbm.at[ip_a.at[1, 0]], rows_b, sem_gb)
        pltpu.make_async_copy(y_hbm.at[ip_a.at[0, 0]], rows_a, sem_ga).wait()
        pltpu.sync_copy(rows_a, spm.at[ip_a.at[0, 1]], add=True)

        pltpu.make_async_copy(ed4.at[w, pl.ds(0, 2)], ip_b, sem_ib).wait()
        pltpu.async_copy(y_hbm.at[ip_b.at[0, 0]], rows_a, sem_ga)
        pltpu.make_async_copy(y_hbm.at[ip_a.at[1, 0]], rows_b, sem_gb).wait()
        pltpu.sync_copy(rows_b, spm.at[ip_a.at[1, 1]], add=True)

        @pl.when(q < Q - 1)
        def _():
            pltpu.async_copy(ed4.at[w, pl.ds(b + 4, 2)], ip_a, sem_ia)

        pltpu.async_copy(y_hbm.at[ip_b.at[1, 0]], rows_b, sem_gb)
        pltpu.make_async_copy(y_hbm.at[ip_b.at[0, 0]], rows_a, sem_ga).wait()
        pltpu.sync_copy(rows_a, spm.at[ip_b.at[0, 1]], add=True)

        @pl.when(q < Q - 1)
        def _():
            pltpu.make_async_copy(ed4.at[w, pl.ds(0, 2)], ip_a, sem_ia).wait()
            pltpu.async_copy(y_hbm.at[ip_a.at[0, 0]], rows_a, sem_ga)

        pltpu.make_async_copy(y_hbm.at[ip_b.at[1, 0]], rows_b, sem_gb).wait()
        pltpu.sync_copy(rows_b, spm.at[ip_b.at[1, 1]], add=True)

        @pl.when(q < Q - 1)
        def _():
            pltpu.async_copy(ed4.at[w, pl.ds(b + 6, 2)], ip_b, sem_ib)

        return carry

    lax.fori_loop(0, Q, quad, 0)

    plsc.subcore_barrier()
    pltpu.sync_copy(spm.at[pl.ds(s * STRIPE, STRIPE)],
                    out.at[c, pl.ds(s * STRIPE, STRIPE)])


def _dinv_block(dp_ref):
    deg = 1.0 + dp_ref[0, :, 0:1] + dp_ref[1, :, 0:1]
    return lax.rsqrt(deg)


def _tc_a_body(x_ref, dp_ref, w_ref, y_ref):
    dinv = _dinv_block(dp_ref)
    xw = jnp.dot(x_ref[...], w_ref[...], preferred_element_type=jnp.float32)
    y_ref[...] = xw * dinv


def _tc_a(x, degp, W1):
    return pl.pallas_call(
        _tc_a_body,
        grid=(NBLK,),
        in_specs=[
            pl.BlockSpec((BLK, D), lambda i: (i, 0)),
            pl.BlockSpec((2, BLK, 8), lambda i: (0, i, 0)),
            pl.BlockSpec((D, H), lambda i: (0, 0)),
        ],
        out_specs=pl.BlockSpec((BLK, H), lambda i: (i, 0)),
        out_shape=jax.ShapeDtypeStruct((N, H), jnp.float32),
    )(x, degp, W1)


def _tc_b_body(ap_ref, y1_ref, dp_ref, w_ref, b_ref, y2_ref):
    dinv = _dinv_block(dp_ref)
    pre = dinv * (ap_ref[0] + ap_ref[1] + y1_ref[...]) + b_ref[...]
    h = jnp.maximum(pre, 0.0)
    hw = jnp.dot(h, w_ref[...], preferred_element_type=jnp.float32)
    y2_ref[...] = hw * dinv


def _tc_b(aggp, y1, degp, W2, b1):
    return pl.pallas_call(
        _tc_b_body,
        grid=(NBLK,),
        in_specs=[
            pl.BlockSpec((2, BLK, H), lambda i: (0, i, 0)),
            pl.BlockSpec((BLK, H), lambda i: (i, 0)),
            pl.BlockSpec((2, BLK, 8), lambda i: (0, i, 0)),
            pl.BlockSpec((H, H), lambda i: (0, 0)),
            pl.BlockSpec((1, H), lambda i: (0, 0)),
        ],
        out_specs=pl.BlockSpec((BLK, H), lambda i: (i, 0)),
        out_shape=jax.ShapeDtypeStruct((N, H), jnp.float32),
    )(aggp, y1, degp, W2, b1)


def _tc_c_body(ap_ref, y2_ref, dp_ref, b_ref, wc_ref, bc_ref,
               batch_ref, logits_ref, pooled_ref, pacc_ref, cacc_ref):
    i = pl.program_id(0)

    @pl.when(i == 0)
    def _():
        pacc_ref[...] = jnp.zeros_like(pacc_ref)
        cacc_ref[...] = jnp.zeros_like(cacc_ref)

    dinv = _dinv_block(dp_ref)
    pre = dinv * (ap_ref[0] + ap_ref[1] + y2_ref[...]) + b_ref[...]
    h2 = jnp.maximum(pre, 0.0)
    b = batch_ref[0, 0, :]
    gids = lax.broadcasted_iota(jnp.int32, (G, BLK), 0)
    onehot = (b[None, :] == gids).astype(jnp.float32)
    pacc_ref[...] += jnp.dot(onehot, h2, preferred_element_type=jnp.float32)
    cacc_ref[...] += jnp.dot(onehot, jnp.ones((BLK, 128), jnp.float32),
                             preferred_element_type=jnp.float32)

    @pl.when(i == NBLK - 1)
    def _():
        cnt = jnp.maximum(cacc_ref[...], 1.0)
        pooled = pacc_ref[...] / cnt
        pooled_ref[...] = pooled
        logits_ref[...] = (
            jnp.dot(pooled, wc_ref[...], preferred_element_type=jnp.float32)
            + bc_ref[...])


def _tc_c(aggp, y2, degp, b2, WcP, bcP, batch3):
    return pl.pallas_call(
        _tc_c_body,
        grid=(NBLK,),
        in_specs=[
            pl.BlockSpec((2, BLK, H), lambda i: (0, i, 0)),
            pl.BlockSpec((BLK, H), lambda i: (i, 0)),
            pl.BlockSpec((2, BLK, 8), lambda i: (0, i, 0)),
            pl.BlockSpec((1, H), lambda i: (0, 0)),
            pl.BlockSpec((H, 128), lambda i: (0, 0)),
            pl.BlockSpec((1, 128), lambda i: (0, 0)),
            pl.BlockSpec((1, 1, BLK), lambda i: (i, 0, 0)),
        ],
        out_specs=[
            pl.BlockSpec((G, 128), lambda i: (0, 0)),
            pl.BlockSpec((G, H), lambda i: (0, 0)),
        ],
        out_shape=[
            jax.ShapeDtypeStruct((G, 128), jnp.float32),
            jax.ShapeDtypeStruct((G, H), jnp.float32),
        ],
        scratch_shapes=[
            pltpu.VMEM((G, H), jnp.float32),
            pltpu.VMEM((G, 128), jnp.float32),
        ],
    )(aggp, y2, degp, b2, WcP, bcP, batch3)


def kernel(x, edge_index, batch, W1, b1, W2, b2, Wc, bc):
    pad = EP - E
    src_p = jnp.concatenate([edge_index[0],
                             jnp.arange(pad, dtype=jnp.int32) % N])
    dst_p = jnp.concatenate([edge_index[1],
                             N + (jnp.arange(pad, dtype=jnp.int32) % 112)])
    src3 = src_p.reshape(NW, NCH, CHUNK)
    dst3 = dst_p.reshape(NW, NCH, CHUNK)
    ed4 = jnp.stack([src3, dst3], axis=2)

    ones128 = jnp.ones((CHUNK, 128), jnp.float32)
    zrows = jnp.zeros((STRIPE, H), jnp.float32)

    deg_k = _deg_kernel_build()
    agg_k = _agg_kernel_build()
    degp = deg_k(ed4, zrows, ones128)[:, :, :8]
    y1 = _tc_a(x, degp, W1)
    agg1 = agg_k(ed4, y1, zrows)
    y2 = _tc_b(agg1, y1, degp, W2, b1.reshape(1, H))
    agg2 = agg_k(ed4, y2, zrows)

    WcP = jnp.pad(Wc, ((0, 0), (0, 128 - C)))
    bcP = jnp.pad(bc, (0, 128 - C)).reshape(1, 128)
    batch3 = batch.reshape(NBLK, 1, BLK)
    logitsP, pooled = _tc_c(agg2, y2, degp,
                            b2.reshape(1, H), WcP, bcP, batch3)
    return logitsP[:, :C], pooled
